# Initial kernel scaffold; baseline (speedup 1.0000x reference)
import jax
import jax.numpy as jnp
from jax.experimental import pallas as pl


def kernel(x, edge_index, batch, W0, b0, a0, W1, b1, a1, W2, b2, a2, W3, b3):
    raise NotImplementedError("write your pallas kernel here")


# speedup vs baseline: 10.8156x; 10.8156x over previous
import functools

import jax
import jax.numpy as jnp
from jax import lax
from jax.experimental import pallas as pl
from jax.experimental.pallas import tpu as pltpu
from jax.experimental.pallas import tpu_sc as plsc

N = 10000
NP = 10240
E = 320000
D_IN = 128
H = 16
OUT = 2
G = 64

NC = 2
NS = 16
NW = NC * NS
EWP = NP
NCH = EWP // 128
EPAD = NW * EWP - E
SH = NP // NS

_MESH = plsc.VectorSubcoreMesh(core_axis_name="c", subcore_axis_name="s")


@functools.partial(
    pl.kernel,
    mesh=_MESH,
    out_type=jax.ShapeDtypeStruct((NC * NP,), jnp.float32),
    scratch_types=[
        pltpu.VMEM((NCH, 128), jnp.int32),
        pltpu.VMEM((128,), jnp.float32),
        pltpu.VMEM((SH,), jnp.float32),
        pltpu.VMEM_SHARED((NP,), jnp.float32),
    ],
)
def _deg_sc(dst_hbm, out_hbm, dst_v, ones_v, seed_v, acc):
    c = lax.axis_index("c")
    s = lax.axis_index("s")
    wid = s * NC + c
    r0 = pl.multiple_of(s * SH, 8)

    pltpu.sync_copy(dst_hbm.at[wid], dst_v)

    seed = jnp.where(c == 0, jnp.float32(1.0), jnp.float32(0.0))

    def fill(ref, n, val):
        def row(i, carry):
            ref[pl.ds(i * 16, 16)] = jnp.full((16,), val, jnp.float32)
            return carry
        lax.fori_loop(0, n // 16, row, 0)

    fill(seed_v, SH, seed)
    pltpu.sync_copy(seed_v, acc.at[pl.ds(r0, SH)])
    fill(ones_v, 128, jnp.float32(1.0))
    plsc.subcore_barrier()

    def chunk(g, carry):
        pltpu.sync_copy(ones_v, acc.at[dst_v.at[g]], add=True)
        return carry
    lax.fori_loop(0, NCH, chunk, 0)

    plsc.subcore_barrier()
    pltpu.sync_copy(acc.at[pl.ds(r0, SH)], out_hbm.at[pl.ds(c * NP + r0, SH)])


_PROP_SCRATCH = (
    [pltpu.VMEM((NCH, 128), jnp.int32),
     pltpu.VMEM((NCH, 128), jnp.int32),
     pltpu.VMEM((H, 128), jnp.float32),
     pltpu.VMEM((NP,), jnp.float32)]
    + [pltpu.VMEM_SHARED((NP,), jnp.float32) for _ in range(H)]
    + [pltpu.VMEM_SHARED((NP,), jnp.float32) for _ in range(H)]
    + [pltpu.SemaphoreType.DMA]
)


@functools.partial(
    pl.kernel,
    mesh=_MESH,
    out_type=jax.ShapeDtypeStruct((NC * H * NP,), jnp.float32),
    scratch_types=_PROP_SCRATCH,
)
def _prop_sc(u_hbm, src_hbm, dst_hbm, out_hbm, src_v, dst_v, colbuf, zbuf, *rest):
    accs = rest[:H]
    ust = rest[H:2 * H]
    sem = rest[2 * H]
    c = lax.axis_index("c")
    s = lax.axis_index("s")
    wid = s * NC + c

    pltpu.sync_copy(src_hbm.at[wid], src_v)
    pltpu.sync_copy(dst_hbm.at[wid], dst_v)

    for k in range(H):
        @pl.when(s == k)
        def _(k=k):
            pltpu.sync_copy(u_hbm.at[pl.ds(k * NP, NP)], ust[k])

            @pl.when(c == 0)
            def _():
                pltpu.sync_copy(u_hbm.at[pl.ds(k * NP, NP)], accs[k])

            @pl.when(c == 1)
            def _():
                def zrow(i, carry):
                    zbuf[pl.ds(i * 16, 16)] = jnp.zeros((16,), jnp.float32)
                    return carry
                lax.fori_loop(0, NP // 16, zrow, 0)
                pltpu.sync_copy(zbuf, accs[k])

    plsc.subcore_barrier()

    def chunk(g, carry):
        src_row = src_v.at[g]
        dst_row = dst_v.at[g]
        descs = [pltpu.async_copy(ust[k].at[src_row], colbuf.at[k], sem)
                 for k in range(H)]
        for d in descs:
            d.wait()
        for k in range(H):
            pltpu.sync_copy(colbuf.at[k], accs[k].at[dst_row], add=True)
        return carry
    lax.fori_loop(0, NCH, chunk, 0)

    plsc.subcore_barrier()
    for k in range(H):
        @pl.when(s == k)
        def _(k=k):
            pltpu.sync_copy(accs[k], out_hbm.at[pl.ds((c * H + k) * NP, NP)])


def _prep_body(x_ref, w0_ref, dp_ref, u_ref, dinv_ref):
    deg = dp_ref[0:1, :] + dp_ref[1:2, :]
    col = lax.broadcasted_iota(jnp.int32, (1, NP), 1)
    dinv = lax.rsqrt(jnp.maximum(deg, 1.0)) * (col < N).astype(jnp.float32)
    dinv_ref[...] = dinv
    t = lax.dot_general(w0_ref[...], x_ref[...], (((0,), (1,)), ((), ())),
                        preferred_element_type=jnp.float32)
    t = jnp.concatenate([t, jnp.zeros((H, NP - N), jnp.float32)], axis=1)
    u_ref[...] = dinv * t


def _trans_body(p_ref, dinv_ref, b_ref, a_ref, w_ref, u_ref):
    dinv = dinv_ref[...]
    h = dinv * (p_ref[0] + p_ref[1]) + b_ref[...]
    h = jnp.where(h >= 0.0, h, a_ref[0, 0] * h)
    u_ref[...] = dinv * lax.dot_general(
        w_ref[...], h, (((0,), (0,)), ((), ())),
        preferred_element_type=jnp.float32)


def _trans3_body(p_ref, dinv_ref, b_ref, a_ref, u_ref):
    dinv = dinv_ref[...]
    h = dinv * (p_ref[0] + p_ref[1]) + b_ref[...]
    h = jnp.where(h >= 0.0, h, a_ref[0, 0] * h)
    u_ref[...] = dinv * h


def _pool_body(p_ref, dinv_ref, batch_ref, w3_ref, b3_ref, out_ref):
    y = dinv_ref[...] * (p_ref[0] + p_ref[1])
    gid = lax.broadcasted_iota(jnp.int32, (G, 1), 0)
    m = (gid == batch_ref[...]).astype(jnp.float32)
    st = lax.dot_general(y, m, (((1,), (1,)), ((), ())),
                         preferred_element_type=jnp.float32)
    cntc = lax.dot_general(m, jnp.ones((NP, 1), jnp.float32),
                           (((1,), (0,)), ((), ())),
                           preferred_element_type=jnp.float32)
    mean = st / jnp.maximum(cntc.reshape(1, G), 1.0)
    res = lax.dot_general(mean, w3_ref[...], (((0,), (0,)), ((), ())),
                          preferred_element_type=jnp.float32)
    out_ref[...] = res + b3_ref[...] * (cntc > 0.0).astype(jnp.float32)


def kernel(x, edge_index, batch, W0, b0, a0, W1, b1, a1, W2, b2, a2, W3, b3):
    srcp = jnp.concatenate(
        [edge_index[0], jnp.zeros((EPAD,), jnp.int32)]).reshape(NW, NCH, 128)
    dstp = jnp.concatenate(
        [edge_index[1], jnp.full((EPAD,), N, jnp.int32)]).reshape(NW, NCH, 128)
    batchp = jnp.concatenate(
        [batch, jnp.full((NP - N,), G, jnp.int32)]).reshape(1, NP)

    degp = _deg_sc(dstp).reshape(NC, NP)

    u0, dinv = pl.pallas_call(
        _prep_body,
        out_shape=(jax.ShapeDtypeStruct((H, NP), jnp.float32),
                   jax.ShapeDtypeStruct((1, NP), jnp.float32)),
    )(x, W0, degp)

    p = _prop_sc(u0.reshape(-1), srcp, dstp).reshape(NC, H, NP)
    u1 = pl.pallas_call(
        _trans_body, out_shape=jax.ShapeDtypeStruct((H, NP), jnp.float32),
    )(p, dinv, b0.reshape(H, 1), a0.reshape(1, 1), W1)

    p = _prop_sc(u1.reshape(-1), srcp, dstp).reshape(NC, H, NP)
    u2 = pl.pallas_call(
        _trans_body, out_shape=jax.ShapeDtypeStruct((H, NP), jnp.float32),
    )(p, dinv, b1.reshape(H, 1), a1.reshape(1, 1), W2)

    p = _prop_sc(u2.reshape(-1), srcp, dstp).reshape(NC, H, NP)
    u3 = pl.pallas_call(
        _trans3_body, out_shape=jax.ShapeDtypeStruct((H, NP), jnp.float32),
    )(p, dinv, b2.reshape(H, 1), a2.reshape(1, 1))

    p = _prop_sc(u3.reshape(-1), srcp, dstp).reshape(NC, H, NP)
    out = pl.pallas_call(
        _pool_body, out_shape=jax.ShapeDtypeStruct((G, OUT), jnp.float32),
    )(p, dinv, batchp, W3, b3.reshape(1, OUT))
    return out


# speedup vs baseline: 15.8749x; 1.4678x over previous
import functools

import jax
import jax.numpy as jnp
from jax import lax
from jax.experimental import pallas as pl
from jax.experimental.pallas import tpu as pltpu
from jax.experimental.pallas import tpu_sc as plsc

N = 10000
NP = 10240
E = 320000
D_IN = 128
H = 16
OUT = 2
G = 64

NC = 2
NS = 16
NW = NC * NS
EWP = NP
NCH = EWP // 128
EPAD = NW * EWP - E
SH = NP // NS

_MESH = plsc.VectorSubcoreMesh(core_axis_name="c", subcore_axis_name="s")


@functools.partial(
    pl.kernel,
    mesh=_MESH,
    out_type=jax.ShapeDtypeStruct((NC * NP,), jnp.float32),
    scratch_types=[
        pltpu.VMEM((NCH, 128), jnp.int32),
        pltpu.VMEM((128,), jnp.float32),
        pltpu.VMEM((SH,), jnp.float32),
        pltpu.VMEM_SHARED((NP,), jnp.float32),
    ],
)
def _deg_sc(dst_hbm, out_hbm, dst_v, ones_v, seed_v, acc):
    c = lax.axis_index("c")
    s = lax.axis_index("s")
    wid = s * NC + c
    r0 = pl.multiple_of(s * SH, 8)

    pltpu.sync_copy(dst_hbm.at[wid], dst_v)

    seed = jnp.where(c == 0, jnp.float32(1.0), jnp.float32(0.0))

    def fill(ref, n, val):
        def row(i, carry):
            ref[pl.ds(i * 16, 16)] = jnp.full((16,), val, jnp.float32)
            return carry
        lax.fori_loop(0, n // 16, row, 0)

    fill(seed_v, SH, seed)
    pltpu.sync_copy(seed_v, acc.at[pl.ds(r0, SH)])
    fill(ones_v, 128, jnp.float32(1.0))
    plsc.subcore_barrier()

    def chunk(g, carry):
        pltpu.sync_copy(ones_v, acc.at[dst_v.at[g]], add=True)
        return carry
    lax.fori_loop(0, NCH, chunk, 0)

    plsc.subcore_barrier()
    pltpu.sync_copy(acc.at[pl.ds(r0, SH)], out_hbm.at[pl.ds(c * NP + r0, SH)])


_PROP_SCRATCH = (
    [pltpu.VMEM((NCH, 128), jnp.int32),
     pltpu.VMEM((NCH, 128), jnp.int32),
     pltpu.VMEM((H, 128), jnp.float32),
     pltpu.VMEM((H, 128), jnp.float32),
     pltpu.VMEM((NP,), jnp.float32)]
    + [pltpu.VMEM_SHARED((NP,), jnp.float32) for _ in range(H)]
    + [pltpu.VMEM_SHARED((NP,), jnp.float32) for _ in range(H)]
    + [pltpu.SemaphoreType.DMA, pltpu.SemaphoreType.DMA]
)


@functools.partial(
    pl.kernel,
    mesh=_MESH,
    out_type=jax.ShapeDtypeStruct((NC * H * NP,), jnp.float32),
    scratch_types=_PROP_SCRATCH,
)
def _prop_sc(u_hbm, src_hbm, dst_hbm, out_hbm, src_v, dst_v, bufa, bufb, zbuf, *rest):
    accs = rest[:H]
    ust = rest[H:2 * H]
    gsem = rest[2 * H]
    ssem = rest[2 * H + 1]
    c = lax.axis_index("c")
    s = lax.axis_index("s")
    wid = s * NC + c

    pltpu.sync_copy(src_hbm.at[wid], src_v)
    pltpu.sync_copy(dst_hbm.at[wid], dst_v)

    for k in range(H):
        @pl.when(s == k)
        def _(k=k):
            pltpu.sync_copy(u_hbm.at[pl.ds(k * NP, NP)], ust[k])

            @pl.when(c == 0)
            def _():
                pltpu.sync_copy(u_hbm.at[pl.ds(k * NP, NP)], accs[k])

            @pl.when(c == 1)
            def _():
                def zrow(i, carry):
                    zbuf[pl.ds(i * 16, 16)] = jnp.zeros((16,), jnp.float32)
                    return carry
                lax.fori_loop(0, NP // 16, zrow, 0)
                pltpu.sync_copy(zbuf, accs[k])

    plsc.subcore_barrier()

    def fire_g(g, buf):
        for k in range(H):
            pltpu.async_copy(ust[k].at[src_v.at[g]], buf.at[k], gsem)

    def wait_g(g, buf):
        for k in range(H):
            pltpu.make_async_copy(ust[k].at[src_v.at[g]], buf.at[k], gsem).wait()

    def fire_s(g, buf):
        for k in range(H):
            pltpu.async_copy(buf.at[k], accs[k].at[dst_v.at[g]], ssem, add=True)

    def wait_s(g, buf):
        for k in range(H):
            pltpu.make_async_copy(buf.at[k], accs[k].at[dst_v.at[g]], ssem).wait()

    fire_g(0, bufa)

    def chunk2(i, carry):
        e = pl.multiple_of(2 * i, 2)
        o = e + 1
        fire_g(o, bufb)
        wait_g(e, bufa)
        fire_s(e, bufa)
        wait_g(o, bufb)
        fire_s(o, bufb)
        wait_s(e, bufa)

        @pl.when(i < NCH // 2 - 1)
        def _():
            fire_g(e + 2, bufa)
        wait_s(o, bufb)
        return carry
    lax.fori_loop(0, NCH // 2, chunk2, 0)

    plsc.subcore_barrier()
    for k in range(H):
        @pl.when(s == k)
        def _(k=k):
            pltpu.sync_copy(accs[k], out_hbm.at[pl.ds((c * H + k) * NP, NP)])


def _prep_body(x_ref, w0_ref, dp_ref, u_ref, dinv_ref):
    deg = dp_ref[0:1, :] + dp_ref[1:2, :]
    col = lax.broadcasted_iota(jnp.int32, (1, NP), 1)
    dinv = lax.rsqrt(jnp.maximum(deg, 1.0)) * (col < N).astype(jnp.float32)
    dinv_ref[...] = dinv
    t = lax.dot_general(w0_ref[...], x_ref[...], (((0,), (1,)), ((), ())),
                        preferred_element_type=jnp.float32)
    t = jnp.concatenate([t, jnp.zeros((H, NP - N), jnp.float32)], axis=1)
    u_ref[...] = dinv * t


def _trans_body(p_ref, dinv_ref, b_ref, a_ref, w_ref, u_ref):
    dinv = dinv_ref[...]
    h = dinv * (p_ref[0] + p_ref[1]) + b_ref[...]
    h = jnp.where(h >= 0.0, h, a_ref[0, 0] * h)
    u_ref[...] = dinv * lax.dot_general(
        w_ref[...], h, (((0,), (0,)), ((), ())),
        preferred_element_type=jnp.float32)


def _trans3_body(p_ref, dinv_ref, b_ref, a_ref, u_ref):
    dinv = dinv_ref[...]
    h = dinv * (p_ref[0] + p_ref[1]) + b_ref[...]
    h = jnp.where(h >= 0.0, h, a_ref[0, 0] * h)
    u_ref[...] = dinv * h


def _pool_body(p_ref, dinv_ref, batch_ref, w3_ref, b3_ref, out_ref):
    y = dinv_ref[...] * (p_ref[0] + p_ref[1])
    gid = lax.broadcasted_iota(jnp.int32, (G, 1), 0)
    m = (gid == batch_ref[...]).astype(jnp.float32)
    st = lax.dot_general(y, m, (((1,), (1,)), ((), ())),
                         preferred_element_type=jnp.float32)
    cntc = lax.dot_general(m, jnp.ones((NP, 1), jnp.float32),
                           (((1,), (0,)), ((), ())),
                           preferred_element_type=jnp.float32)
    mean = st / jnp.maximum(cntc.reshape(1, G), 1.0)
    res = lax.dot_general(mean, w3_ref[...], (((0,), (0,)), ((), ())),
                          preferred_element_type=jnp.float32)
    out_ref[...] = res + b3_ref[...] * (cntc > 0.0).astype(jnp.float32)


def kernel(x, edge_index, batch, W0, b0, a0, W1, b1, a1, W2, b2, a2, W3, b3):
    srcp = jnp.concatenate(
        [edge_index[0], jnp.zeros((EPAD,), jnp.int32)]).reshape(NW, NCH, 128)
    dstp = jnp.concatenate(
        [edge_index[1], jnp.full((EPAD,), N, jnp.int32)]).reshape(NW, NCH, 128)
    batchp = jnp.concatenate(
        [batch, jnp.full((NP - N,), G, jnp.int32)]).reshape(1, NP)

    degp = _deg_sc(dstp).reshape(NC, NP)

    u0, dinv = pl.pallas_call(
        _prep_body,
        out_shape=(jax.ShapeDtypeStruct((H, NP), jnp.float32),
                   jax.ShapeDtypeStruct((1, NP), jnp.float32)),
    )(x, W0, degp)

    p = _prop_sc(u0.reshape(-1), srcp, dstp).reshape(NC, H, NP)
    u1 = pl.pallas_call(
        _trans_body, out_shape=jax.ShapeDtypeStruct((H, NP), jnp.float32),
    )(p, dinv, b0.reshape(H, 1), a0.reshape(1, 1), W1)

    p = _prop_sc(u1.reshape(-1), srcp, dstp).reshape(NC, H, NP)
    u2 = pl.pallas_call(
        _trans_body, out_shape=jax.ShapeDtypeStruct((H, NP), jnp.float32),
    )(p, dinv, b1.reshape(H, 1), a1.reshape(1, 1), W2)

    p = _prop_sc(u2.reshape(-1), srcp, dstp).reshape(NC, H, NP)
    u3 = pl.pallas_call(
        _trans3_body, out_shape=jax.ShapeDtypeStruct((H, NP), jnp.float32),
    )(p, dinv, b2.reshape(H, 1), a2.reshape(1, 1))

    p = _prop_sc(u3.reshape(-1), srcp, dstp).reshape(NC, H, NP)
    out = pl.pallas_call(
        _pool_body, out_shape=jax.ShapeDtypeStruct((G, OUT), jnp.float32),
    )(p, dinv, batchp, W3, b3.reshape(1, OUT))
    return out


# speedup vs baseline: 26.8840x; 1.6935x over previous
import functools

import jax
import jax.numpy as jnp
from jax import lax
from jax.experimental import pallas as pl
from jax.experimental.pallas import tpu as pltpu
from jax.experimental.pallas import tpu_sc as plsc

N = 10000
NP = 10240
E = 320000
D_IN = 128
H = 16
OUT = 2
G = 64

NC = 2
NS = 16
NW = NC * NS
EWP = NP
NCH = EWP // 128
EPAD = NW * EWP - E
SH = NP // NS

_MESH = plsc.VectorSubcoreMesh(core_axis_name="c", subcore_axis_name="s")


@functools.partial(
    pl.kernel,
    mesh=_MESH,
    out_type=jax.ShapeDtypeStruct((NC * NP,), jnp.float32),
    scratch_types=[
        pltpu.VMEM((NCH, 128), jnp.int32),
        pltpu.VMEM((128,), jnp.float32),
        pltpu.VMEM((SH,), jnp.float32),
        pltpu.VMEM_SHARED((NP,), jnp.float32),
    ],
)
def _deg_sc(dst_hbm, out_hbm, dst_v, ones_v, seed_v, acc):
    c = lax.axis_index("c")
    s = lax.axis_index("s")
    wid = s * NC + c
    r0 = pl.multiple_of(s * SH, 8)

    pltpu.sync_copy(dst_hbm.at[wid], dst_v)

    seed = jnp.where(c == 0, jnp.float32(1.0), jnp.float32(0.0))

    def fill(ref, n, val):
        def row(i, carry):
            ref[pl.ds(i * 16, 16)] = jnp.full((16,), val, jnp.float32)
            return carry
        lax.fori_loop(0, n // 16, row, 0)

    fill(seed_v, SH, seed)
    pltpu.sync_copy(seed_v, acc.at[pl.ds(r0, SH)])
    fill(ones_v, 128, jnp.float32(1.0))
    plsc.subcore_barrier()

    def chunk(g, carry):
        pltpu.sync_copy(ones_v, acc.at[dst_v.at[g]], add=True)
        return carry
    lax.fori_loop(0, NCH, chunk, 0)

    plsc.subcore_barrier()
    pltpu.sync_copy(acc.at[pl.ds(r0, SH)], out_hbm.at[pl.ds(c * NP + r0, SH)])


CE = 8192
EP2 = 163840
NCHE = EP2 // CE
EPAD2 = EP2 - E // NC

_PROP_SCRATCH = [
    pltpu.VMEM((CE,), jnp.int32),
    pltpu.VMEM((CE,), jnp.int32),
    pltpu.VMEM((CE,), jnp.int32),
    pltpu.VMEM((CE,), jnp.int32),
    pltpu.VMEM((NP,), jnp.float32),
    pltpu.VMEM((NP,), jnp.float32),
    pltpu.SemaphoreType.DMA,
]


@functools.partial(
    pl.kernel,
    mesh=_MESH,
    out_type=jax.ShapeDtypeStruct((NC * H * NP,), jnp.float32),
    compiler_params=pltpu.CompilerParams(needs_layout_passes=False),
    scratch_types=_PROP_SCRATCH,
)
def _prop_sc(u_hbm, src_hbm, dst_hbm, out_hbm,
             srca, dsta, srcb, dstb, uk, acc, sem):
    c = lax.axis_index("c")
    k = lax.axis_index("s")
    col0 = pl.multiple_of(k * NP, 8)
    base = pl.multiple_of(c * EP2, 8)

    pltpu.sync_copy(u_hbm.at[pl.ds(col0, NP)], uk)

    @pl.when(c == 0)
    def _():
        pltpu.sync_copy(u_hbm.at[pl.ds(col0, NP)], acc)

    @pl.when(c == 1)
    def _():
        def zrow(i, carry):
            acc[pl.ds(i * 16, 16)] = jnp.zeros((16,), jnp.float32)
            return carry
        lax.fori_loop(0, NP // 16, zrow, 0)

    def fire(j, sbuf, dbuf):
        pltpu.async_copy(src_hbm.at[pl.ds(base + j * CE, CE)], sbuf, sem)
        pltpu.async_copy(dst_hbm.at[pl.ds(base + j * CE, CE)], dbuf, sem)

    def wait(j, sbuf, dbuf):
        pltpu.make_async_copy(src_hbm.at[pl.ds(base + j * CE, CE)], sbuf, sem).wait()
        pltpu.make_async_copy(dst_hbm.at[pl.ds(base + j * CE, CE)], dbuf, sem).wait()

    def compute(sbuf, dbuf):
        def body(j, carry):
            idxs = sbuf[pl.ds(j * 16, 16)]
            didx = dbuf[pl.ds(j * 16, 16)]
            vals = plsc.load_gather(uk, [idxs])
            plsc.addupdate_scatter(acc, [didx], vals)
            return carry
        lax.fori_loop(0, CE // 16, body, 0)

    fire(0, srca, dsta)

    def pair(i, carry):
        e = pl.multiple_of(2 * i, 2)
        o = e + 1
        fire(o, srcb, dstb)
        wait(e, srca, dsta)
        compute(srca, dsta)

        @pl.when(i < NCHE // 2 - 1)
        def _():
            fire(e + 2, srca, dsta)
        wait(o, srcb, dstb)
        compute(srcb, dstb)
        return carry
    lax.fori_loop(0, NCHE // 2, pair, 0)

    pltpu.sync_copy(acc, out_hbm.at[pl.ds(pl.multiple_of((c * H + k) * NP, 8), NP)])


def _prep_body(x_ref, w0_ref, dp_ref, u_ref, dinv_ref):
    deg = dp_ref[0:1, :] + dp_ref[1:2, :]
    col = lax.broadcasted_iota(jnp.int32, (1, NP), 1)
    dinv = lax.rsqrt(jnp.maximum(deg, 1.0)) * (col < N).astype(jnp.float32)
    dinv_ref[...] = dinv
    t = lax.dot_general(w0_ref[...], x_ref[...], (((0,), (1,)), ((), ())),
                        preferred_element_type=jnp.float32)
    t = jnp.concatenate([t, jnp.zeros((H, NP - N), jnp.float32)], axis=1)
    u_ref[...] = dinv * t


def _trans_body(p_ref, dinv_ref, b_ref, a_ref, w_ref, u_ref):
    dinv = dinv_ref[...]
    h = dinv * (p_ref[0] + p_ref[1]) + b_ref[...]
    h = jnp.where(h >= 0.0, h, a_ref[0, 0] * h)
    u_ref[...] = dinv * lax.dot_general(
        w_ref[...], h, (((0,), (0,)), ((), ())),
        preferred_element_type=jnp.float32)


def _trans3_body(p_ref, dinv_ref, b_ref, a_ref, u_ref):
    dinv = dinv_ref[...]
    h = dinv * (p_ref[0] + p_ref[1]) + b_ref[...]
    h = jnp.where(h >= 0.0, h, a_ref[0, 0] * h)
    u_ref[...] = dinv * h


def _pool_body(p_ref, dinv_ref, batch_ref, w3_ref, b3_ref, out_ref):
    y = dinv_ref[...] * (p_ref[0] + p_ref[1])
    gid = lax.broadcasted_iota(jnp.int32, (G, 1), 0)
    m = (gid == batch_ref[...]).astype(jnp.float32)
    st = lax.dot_general(y, m, (((1,), (1,)), ((), ())),
                         preferred_element_type=jnp.float32)
    cntc = lax.dot_general(m, jnp.ones((NP, 1), jnp.float32),
                           (((1,), (0,)), ((), ())),
                           preferred_element_type=jnp.float32)
    mean = st / jnp.maximum(cntc.reshape(1, G), 1.0)
    res = lax.dot_general(mean, w3_ref[...], (((0,), (0,)), ((), ())),
                          preferred_element_type=jnp.float32)
    out_ref[...] = res + b3_ref[...] * (cntc > 0.0).astype(jnp.float32)


def kernel(x, edge_index, batch, W0, b0, a0, W1, b1, a1, W2, b2, a2, W3, b3):
    dstp3 = jnp.concatenate(
        [edge_index[1], jnp.full((EPAD,), N, jnp.int32)]).reshape(NW, NCH, 128)
    batchp = jnp.concatenate(
        [batch, jnp.full((NP - N,), G, jnp.int32)]).reshape(1, NP)

    eh = E // NC
    pad = jnp.zeros((EPAD2,), jnp.int32)
    padn = jnp.full((EPAD2,), N, jnp.int32)
    srcp = jnp.concatenate(
        [edge_index[0, :eh], pad, edge_index[0, eh:], pad])
    dstp = jnp.concatenate(
        [edge_index[1, :eh], padn, edge_index[1, eh:], padn])

    degp = _deg_sc(dstp3).reshape(NC, NP)

    u0, dinv = pl.pallas_call(
        _prep_body,
        out_shape=(jax.ShapeDtypeStruct((H, NP), jnp.float32),
                   jax.ShapeDtypeStruct((1, NP), jnp.float32)),
    )(x, W0, degp)

    p = _prop_sc(u0.reshape(-1), srcp, dstp).reshape(NC, H, NP)
    u1 = pl.pallas_call(
        _trans_body, out_shape=jax.ShapeDtypeStruct((H, NP), jnp.float32),
    )(p, dinv, b0.reshape(H, 1), a0.reshape(1, 1), W1)

    p = _prop_sc(u1.reshape(-1), srcp, dstp).reshape(NC, H, NP)
    u2 = pl.pallas_call(
        _trans_body, out_shape=jax.ShapeDtypeStruct((H, NP), jnp.float32),
    )(p, dinv, b1.reshape(H, 1), a1.reshape(1, 1), W2)

    p = _prop_sc(u2.reshape(-1), srcp, dstp).reshape(NC, H, NP)
    u3 = pl.pallas_call(
        _trans3_body, out_shape=jax.ShapeDtypeStruct((H, NP), jnp.float32),
    )(p, dinv, b2.reshape(H, 1), a2.reshape(1, 1))

    p = _prop_sc(u3.reshape(-1), srcp, dstp).reshape(NC, H, NP)
    out = pl.pallas_call(
        _pool_body, out_shape=jax.ShapeDtypeStruct((G, OUT), jnp.float32),
    )(p, dinv, batchp, W3, b3.reshape(1, OUT))
    return out
